# baseline (speedup 1.0000x reference)
import functools

import jax
import jax.numpy as jnp
from jax import lax
from jax.experimental import pallas as pl
from jax.experimental.pallas import tpu as pltpu
from jax.experimental.pallas import tpu_sc as plsc

VOCAB = 1000000
D = 32
B = 16384
L = 200

_NC = 2
_NS = 16
_NW = _NC * _NS
_VTILES = VOCAB // 128
_NT = _VTILES + 1
_TAIL_W = _VTILES % _NW
_LB = L // 8
_BB = B // 128
_PAIRS = _LB * _BB
_PER_W = _PAIRS // _NW
_NTILES_OUT = L * (D // 8) * _BB

_MESH = dict(core_axis_name="c", subcore_axis_name="s")


_TCBLKV = 8192


def _make_detile():
    def body(wt_ref, out_ref):
        blk = wt_ref[...]
        t = blk.T
        t3 = t.reshape(_TCBLKV // 4, 4, D)
        for q in range(4):
            out_ref[:, q * D:(q + 1) * D] = t3[:, q, :]

    return pl.pallas_call(
        body,
        grid=(pl.cdiv(VOCAB, _TCBLKV),),
        in_specs=[pl.BlockSpec((D, _TCBLKV), lambda j: (0, j))],
        out_specs=pl.BlockSpec((_TCBLKV // 4, 128), lambda j: (j, 0)),
        out_shape=jax.ShapeDtypeStruct((VOCAB * D // 128, 128), jnp.float32),
    )


def _make_gather():
    @functools.partial(
        pl.kernel,
        mesh=plsc.VectorSubcoreMesh(**_MESH),
        compiler_params=pltpu.CompilerParams(
            use_tc_tiling_on_sc=False, needs_layout_passes=False
        ),
        out_type=jax.ShapeDtypeStruct((_NTILES_OUT, 8, 128), jnp.float32),
        scratch_types=[
            pltpu.VMEM((2, 8, 128), jnp.int32),
            pltpu.VMEM((4, 256, D), jnp.float32),
            pltpu.VMEM((2, 2, D, 129), jnp.float32),
            pltpu.SemaphoreType.DMA,
            pltpu.SemaphoreType.DMA,
            pltpu.SemaphoreType.DMA,
        ],
    )
    def gat(tbl_hbm, xt_hbm, o_hbm, idx2, rows2, outt2, sem_i, sem_g, sem_w):
        wid = lax.axis_index("s") * _NC + lax.axis_index("c")
        iota = lax.iota(jnp.int32, 16)
        rows_t = [16 * g + iota for g in range(2)]
        total = _PER_W * 4
        _LOOKAHEAD = 3

        pltpu.sync_copy(xt_hbm.at[wid // _BB, wid % _BB], idx2.at[0])
        for s0 in range(_LOOKAHEAD):
            for li in range(2):
                pltpu.async_copy(
                    tbl_hbm.at[idx2.at[0, 2 * s0 + li]],
                    rows2.at[s0, pl.ds(li * 128, 128)],
                    sem_g,
                )

        def step(s, carry):
            k = s // 4
            h = s % 4
            p = wid + k * _NW
            lb = p // _BB
            bb = p % _BB

            @pl.when((h == 0) & (k + 1 < _PER_W))
            def _():
                pn = wid + (k + 1) * _NW
                pltpu.async_copy(
                    xt_hbm.at[pn // _BB, pn % _BB],
                    idx2.at[(k + 1) % 2],
                    sem_i,
                )

            @pl.when(s + _LOOKAHEAD < total)
            def _():
                sn = s + _LOOKAHEAD
                kn = sn // 4
                hn = sn % 4

                @pl.when(hn == 0)
                def _():
                    pltpu.make_async_copy(
                        o_hbm.at[0], idx2.at[0], sem_i
                    ).wait()

                for li in range(2):
                    pltpu.async_copy(
                        tbl_hbm.at[idx2.at[kn % 2, 2 * hn + li]],
                        rows2.at[sn % 4, pl.ds(li * 128, 128)],
                        sem_g,
                    )

            pltpu.make_async_copy(o_hbm.at[0], rows2.at[0], sem_g).wait()

            @pl.when(s >= 2)
            def _():
                pltpu.make_async_copy(o_hbm.at[0], rows2.at[0], sem_w).wait()

            for li in range(2):
                for b in range(128):
                    for g in range(2):
                        v = rows2[s % 4, li * 128 + b, pl.ds(g * 16, 16)]
                        plsc.store_scatter(
                            outt2.at[s % 2, li], [rows_t[g], iota * 0 + b], v
                        )
                row0 = (lb * 8 + h * 2 + li) * 4 * _BB + bb
                for db in range(4):
                    pltpu.async_copy(
                        outt2.at[s % 2, li, pl.ds(8 * db, 8), pl.ds(0, 128)],
                        o_hbm.at[row0 + db * _BB],
                        sem_w,
                    )
            return carry

        lax.fori_loop(0, total, step, 0)

        pltpu.make_async_copy(o_hbm.at[0], rows2.at[0], sem_w).wait()
        pltpu.make_async_copy(o_hbm.at[0], rows2.at[0], sem_w).wait()

    return gat


_detile = _make_detile()
_gather = _make_gather()


def kernel(x, weight):
    wt = weight.T
    tbl4 = _detile(wt)
    tbl = tbl4.reshape(VOCAB, D)
    xq = x.T.reshape(_LB, 8, _BB, 128).transpose(0, 2, 1, 3)
    t = _gather(tbl, xq)
    t5 = t.reshape(L, D // 8, _BB, 8, 128)
    return t5.transpose(2, 4, 0, 1, 3).reshape(B, L, D)


# speedup vs baseline: 1.0941x; 1.0941x over previous
import functools

import jax
import jax.numpy as jnp
from jax import lax
from jax.experimental import pallas as pl
from jax.experimental.pallas import tpu as pltpu
from jax.experimental.pallas import tpu_sc as plsc

VOCAB = 1000000
D = 32
B = 16384
L = 200

_NC = 2
_NS = 16
_NW = _NC * _NS
_VTILES = VOCAB // 128
_NT = _VTILES + 1
_TAIL_W = _VTILES % _NW
_LB = L // 8
_BB = B // 128
_PAIRS = _LB * _BB
_PER_W = _PAIRS // _NW
_NTILES_OUT = L * (D // 8) * _BB

_MESH = dict(core_axis_name="c", subcore_axis_name="s")


_TCBLKV = 8192


def _make_detile():
    def body(wt_ref, out_ref):
        blk = wt_ref[...]
        t = blk.T
        t3 = t.reshape(_TCBLKV // 4, 4, D)
        for q in range(4):
            out_ref[:, q * D:(q + 1) * D] = t3[:, q, :]

    return pl.pallas_call(
        body,
        grid=(pl.cdiv(VOCAB, _TCBLKV),),
        in_specs=[pl.BlockSpec((D, _TCBLKV), lambda j: (0, j))],
        out_specs=pl.BlockSpec((_TCBLKV // 4, 128), lambda j: (j, 0)),
        out_shape=jax.ShapeDtypeStruct((VOCAB * D // 128, 128), jnp.float32),
    )


def _make_gather():
    @functools.partial(
        pl.kernel,
        mesh=plsc.VectorSubcoreMesh(**_MESH),
        compiler_params=pltpu.CompilerParams(
            use_tc_tiling_on_sc=False, needs_layout_passes=False
        ),
        out_type=jax.ShapeDtypeStruct((_NTILES_OUT, 8, 128), jnp.float32),
        scratch_types=[
            pltpu.VMEM((2, 8, 128), jnp.int32),
            pltpu.VMEM((4, 128, D), jnp.float32),
            pltpu.VMEM((2, D, 129), jnp.float32),
            pltpu.SemaphoreType.DMA,
            pltpu.SemaphoreType.DMA,
            pltpu.SemaphoreType.DMA,
        ],
    )
    def gat(tbl_hbm, xt_hbm, o_hbm, idx2, rows2, outt2, sem_i, sem_g, sem_w):
        wid = lax.axis_index("s") * _NC + lax.axis_index("c")
        iota = lax.iota(jnp.int32, 16)
        rows_t = [16 * g + iota for g in range(2)]
        total = _PER_W * 8
        _LOOKAHEAD = 3

        pltpu.sync_copy(xt_hbm.at[wid // _BB, wid % _BB], idx2.at[0])
        for s0 in range(_LOOKAHEAD):
            pltpu.async_copy(
                tbl_hbm.at[idx2.at[0, s0]], rows2.at[s0], sem_g
            )

        def step(s, carry):
            k = s // 8
            q = s % 8
            p = wid + k * _NW
            lb = p // _BB
            bb = p % _BB

            @pl.when((q == 0) & (k + 1 < _PER_W))
            def _():
                pn = wid + (k + 1) * _NW
                pltpu.async_copy(
                    xt_hbm.at[pn // _BB, pn % _BB],
                    idx2.at[(k + 1) % 2],
                    sem_i,
                )

            @pl.when(s + _LOOKAHEAD < total)
            def _():
                sn = s + _LOOKAHEAD
                kn = sn // 8
                qn = sn % 8

                @pl.when(qn == 0)
                def _():
                    pltpu.make_async_copy(
                        o_hbm.at[0], idx2.at[0], sem_i
                    ).wait()

                pltpu.async_copy(
                    tbl_hbm.at[idx2.at[kn % 2, qn]], rows2.at[sn % 4], sem_g
                )

            pltpu.make_async_copy(o_hbm.at[0], rows2.at[0], sem_g).wait()

            @pl.when(s >= 2)
            def _():
                pltpu.make_async_copy(o_hbm.at[0], rows2.at[0], sem_w).wait()

            for b in range(128):
                for g in range(2):
                    v = rows2[s % 4, b, pl.ds(g * 16, 16)]
                    plsc.store_scatter(
                        outt2.at[s % 2], [rows_t[g], iota * 0 + b], v
                    )

            row0 = (lb * 8 + q) * 4 * _BB + bb
            for db in range(4):
                pltpu.async_copy(
                    outt2.at[s % 2, pl.ds(8 * db, 8), pl.ds(0, 128)],
                    o_hbm.at[row0 + db * _BB],
                    sem_w,
                )
            return carry

        lax.fori_loop(0, total, step, 0)

        pltpu.make_async_copy(o_hbm.at[0], rows2.at[0], sem_w).wait()
        pltpu.make_async_copy(o_hbm.at[0], rows2.at[0], sem_w).wait()

    return gat


_detile = _make_detile()
_gather = _make_gather()


def kernel(x, weight):
    wt = weight.T
    tbl4 = _detile(wt)
    tbl = tbl4.reshape(VOCAB, D)
    xq = x.T.reshape(_LB, 8, _BB, 128).transpose(0, 2, 1, 3)
    t = _gather(tbl, xq)
    t5 = t.reshape(L, D // 8, _BB, 8, 128)
    return t5.transpose(2, 4, 0, 1, 3).reshape(B, L, D)
